# baseline (speedup 1.0000x reference)
import jax
import jax.numpy as jnp
from jax.experimental import pallas as pl
from jax.experimental.pallas import tpu as pltpu

N = 4096
CMAX = 512
_SIZES = [128, 384] + [512] * 6 + [384, 128]
_STARTS = [sum(_SIZES[:k]) for k in range(len(_SIZES))]
CHUNKS = list(zip(_STARTS, _SIZES))


def _body(q_hbm, at_hbm, xt_ref, yt_ref, c_ref, out_ref, qbuf, abuf, sem):
    def start(k):
        r0, sz = CHUNKS[k]
        slot = k % 2
        cq = pltpu.make_async_copy(
            q_hbm.at[pl.ds(r0, sz)], qbuf.at[slot, pl.ds(0, sz)],
            sem.at[slot])
        ca = pltpu.make_async_copy(
            at_hbm.at[pl.ds(r0, sz)], abuf.at[slot, pl.ds(0, sz)],
            sem.at[slot])
        cq.start()
        ca.start()
        return cq, ca

    descs = [start(0)]
    m = None
    for k, (r0, sz) in enumerate(CHUNKS):
        if k + 1 < len(CHUNKS):
            descs.append(start(k + 1))
        cq, ca = descs[k]
        cq.wait()
        ca.wait()
        slot = k % 2
        s = (jnp.sum(qbuf[slot, :sz, :] * xt_ref[...], axis=1)
             + jnp.sum(abuf[slot, :sz, :] * yt_ref[...], axis=1))
        pg = s + c_ref[0, r0:r0 + sz]
        mk = jnp.max(jnp.abs(pg))
        m = mk if m is None else jnp.maximum(m, mk)
    out_ref[0, 0] = m / (1.0 + jnp.max(jnp.abs(c_ref[...])))


def kernel(Q, AT, b, c, x, y, Iy, il, iu, l, u):
    xt = x.reshape(1, N)
    yt = y.reshape(1, N)
    crow = c.reshape(1, N)
    out = pl.pallas_call(
        _body,
        in_specs=[
            pl.BlockSpec(memory_space=pl.ANY),
            pl.BlockSpec(memory_space=pl.ANY),
            pl.BlockSpec((1, N), lambda: (0, 0)),
            pl.BlockSpec((1, N), lambda: (0, 0)),
            pl.BlockSpec((1, N), lambda: (0, 0)),
        ],
        out_specs=pl.BlockSpec(memory_space=pltpu.SMEM),
        out_shape=jax.ShapeDtypeStruct((1, 1), jnp.float32),
        scratch_shapes=[
            pltpu.VMEM((2, CMAX, N), jnp.float32),
            pltpu.VMEM((2, CMAX, N), jnp.float32),
            pltpu.SemaphoreType.DMA((2,)),
        ],
    )(Q, AT, xt, yt, crow)
    return out[0, 0]


# speedup vs baseline: 1.0240x; 1.0240x over previous
import jax
import jax.numpy as jnp
from jax.experimental import pallas as pl
from jax.experimental.pallas import tpu as pltpu

N = 4096
BM = 256


def _body(q_ref, at_ref, xt_ref, yt_ref, c_ref, out_ref, gmax_ref, cmax_ref):
    i = pl.program_id(0)
    qx = jnp.sum(q_ref[...] * xt_ref[...], axis=1)
    aty = jnp.sum(at_ref[...] * yt_ref[...], axis=1)
    pg = qx + aty + c_ref[0, pl.ds(i * BM, BM)]
    m = jnp.max(jnp.abs(pg))

    @pl.when(i == 0)
    def _init():
        gmax_ref[0, 0] = m
        cmax_ref[0, 0] = jnp.max(jnp.abs(c_ref[...]))

    @pl.when(i > 0)
    def _acc():
        gmax_ref[0, 0] = jnp.maximum(gmax_ref[0, 0], m)

    @pl.when(i == pl.num_programs(0) - 1)
    def _fin():
        out_ref[0, 0] = gmax_ref[0, 0] / (1.0 + cmax_ref[0, 0])


def kernel(Q, AT, b, c, x, y, Iy, il, iu, l, u):
    xt = x.reshape(1, N)
    yt = y.reshape(1, N)
    crow = c.reshape(1, N)
    grid = N // BM
    out = pl.pallas_call(
        _body,
        grid=(grid,),
        in_specs=[
            pl.BlockSpec((BM, N), lambda i: (i, 0)),
            pl.BlockSpec((BM, N), lambda i: (i, 0)),
            pl.BlockSpec((1, N), lambda i: (0, 0)),
            pl.BlockSpec((1, N), lambda i: (0, 0)),
            pl.BlockSpec((1, N), lambda i: (0, 0)),
        ],
        out_specs=pl.BlockSpec(memory_space=pltpu.SMEM),
        out_shape=jax.ShapeDtypeStruct((1, 1), jnp.float32),
        scratch_shapes=[
            pltpu.SMEM((1, 1), jnp.float32),
            pltpu.SMEM((1, 1), jnp.float32),
        ],
    )(Q, AT, xt, yt, crow)
    return out[0, 0]
